# baseline (speedup 1.0000x reference)
import functools

import jax
import jax.numpy as jnp
from jax import lax
from jax.experimental import pallas as pl
from jax.experimental.pallas import tpu as pltpu
from jax.experimental.pallas import tpu_sc as plsc

_N = 10000
_E = 320000
_R = 8
_NB = 4
_GRID = 100

_NPAD = 10112
_NW = 32
_EPT = 10240
_EP = _EPT * _NW
_CH = 128
_NCHUNK = _EPT // _CH
_RSUB = _NPAD // 16

def _mesh():
    return plsc.VectorSubcoreMesh(core_axis_name="c", subcore_axis_name="s",
                                  num_cores=2, num_subcores=16)


def _sc_deg_gidx(src3, ety3, dst3, zeros16):
    @functools.partial(
        pl.kernel,
        out_type=(
            jax.ShapeDtypeStruct((_NW, _NCHUNK, _CH), jnp.int32),
            jax.ShapeDtypeStruct((2, _NPAD, 16), jnp.float32),
        ),
        mesh=_mesh(),
        compiler_params=pltpu.CompilerParams(use_tc_tiling_on_sc=False),
        scratch_types=[
            pltpu.VMEM((_NCHUNK, _CH), jnp.int32),
            pltpu.VMEM((_NCHUNK, _CH), jnp.int32),
            pltpu.VMEM((_NCHUNK, _CH), jnp.int32),
            pltpu.VMEM((_NCHUNK, _CH), jnp.int32),
            pltpu.VMEM((_CH, 16), jnp.float32),
            pltpu.VMEM_SHARED((_NPAD, 16), jnp.float32),
            pltpu.SemaphoreType.DMA,
        ],
    )
    def k(src_h, ety_h, dst_h, z16_h, gidx_h, deg_h,
          src_v, ety_v, gidx_v, dst_v, ones_v, dacc, sem):
        c = lax.axis_index("c")
        s = lax.axis_index("s")
        wid = s * 2 + c
        rows = pl.ds(s * _RSUB, _RSUB)
        pltpu.sync_copy(z16_h.at[rows], dacc.at[rows])
        pltpu.sync_copy(src_h.at[wid], src_v)
        pltpu.sync_copy(ety_h.at[wid], ety_v)
        pltpu.sync_copy(dst_h.at[wid], dst_v)
        one16 = jnp.ones((16,), jnp.float32)

        def fill(i, _):
            ones_v[i, :] = one16
            return 0

        lax.fori_loop(0, _CH, fill, 0)

        def gix(i, _):
            for j in range(_CH // 16):
                sl = pl.ds(j * 16, 16)
                gidx_v[i, sl] = ety_v[i, sl] * _NPAD + src_v[i, sl]
            return 0

        lax.fori_loop(0, _NCHUNK, gix, 0)
        pltpu.sync_copy(gidx_v, gidx_h.at[wid])
        plsc.subcore_barrier()

        def step(g, _):
            pltpu.sync_copy(ones_v, dacc.at[dst_v.at[g]], add=True)
            return 0

        lax.fori_loop(0, _NCHUNK, step, 0)
        plsc.subcore_barrier()
        pltpu.sync_copy(dacc.at[rows], deg_h.at[c, rows])

    return k(src3, ety3, dst3, zeros16)


_NBUF = 4
_TOTCH = _EP // _CH
_CNT0 = 136
_CNT1 = (_TOTCH - 16 * _CNT0) // 16
_CNTMAX = max(_CNT0, _CNT1)


def _sc_agg(z, gidx3, dst3, zeros_d, dout):
    @functools.partial(
        pl.kernel,
        out_type=jax.ShapeDtypeStruct((2, _NPAD, dout), jnp.float32),
        mesh=_mesh(),
        compiler_params=pltpu.CompilerParams(use_tc_tiling_on_sc=False),
        scratch_types=[
            pltpu.VMEM((_CNTMAX, _CH), jnp.int32),
            pltpu.VMEM((_CNTMAX, _CH), jnp.int32),
            [pltpu.VMEM((_CH, dout), jnp.float32)] * _NBUF,
            [pltpu.SemaphoreType.DMA] * _NBUF,
            pltpu.VMEM_SHARED((_NPAD, dout), jnp.float32),
        ],
    )
    def k(z_h, gidx_h, dst_h, zz_h, agg_h, gidx_v, dst_v, rows_v, sems, acc):
        c = lax.axis_index("c")
        s = lax.axis_index("s")
        rows = pl.ds(s * _RSUB, _RSUB)
        pltpu.sync_copy(zz_h.at[rows], acc.at[rows])

        def gather(g, b):
            pltpu.async_copy(z_h.at[gidx_v.at[g]], rows_v[b], sems[b])

        def gwait(g, b):
            pltpu.make_async_copy(z_h.at[gidx_v.at[g]], rows_v[b],
                                  sems[b]).wait()

        def run(start, cnt):
            if cnt > 0:
                span = pl.ds(start, cnt)
                head = pl.ds(0, cnt)
                pltpu.sync_copy(gidx_h.at[span], gidx_v.at[head])
                pltpu.sync_copy(dst_h.at[span], dst_v.at[head])
            plsc.subcore_barrier()
            if cnt > 0:
                for b in range(_NBUF - 1):
                    gather(b, b)

                def outer(t, _):
                    for b in range(_NBUF):
                        g = t * _NBUF + b
                        nxt = g + _NBUF - 1
                        bn = (b + _NBUF - 1) % _NBUF

                        @pl.when(nxt < cnt)
                        def _():
                            gather(nxt, bn)

                        gwait(g, b)
                        pltpu.sync_copy(rows_v[b], acc.at[dst_v.at[g]],
                                        add=True)
                    return 0

                lax.fori_loop(0, cnt // _NBUF, outer, 0)
            plsc.subcore_barrier()
            pltpu.sync_copy(acc.at[rows], agg_h.at[c, rows])

        @pl.when(c == 0)
        def _():
            run(s * _CNT0, _CNT0)

        @pl.when(c == 1)
        def _():
            run(16 * _CNT0 + s * _CNT1, _CNT1)

    return k(z, gidx3.reshape(_TOTCH, _CH), dst3.reshape(_TOTCH, _CH),
             zeros_d)


def _tc_z(x, bases, comb, din, dout):
    nblk = 8
    br = _NPAD // nblk

    def body(x_ref, b_ref, c_ref, z_ref):
        w = (c_ref[0, 0, 0] * b_ref[0] + c_ref[0, 0, 1] * b_ref[1]
             + c_ref[0, 0, 2] * b_ref[2] + c_ref[0, 0, 3] * b_ref[3])
        z_ref[...] = jnp.dot(x_ref[...], w, preferred_element_type=jnp.float32)

    return pl.pallas_call(
        body,
        grid=(_R, nblk),
        in_specs=[
            pl.BlockSpec((br, din), lambda r, nb: (nb, 0)),
            pl.BlockSpec((_NB, din, dout), lambda r, nb: (0, 0, 0)),
            pl.BlockSpec((1, 1, _NB), lambda r, nb: (r, 0, 0)),
        ],
        out_specs=pl.BlockSpec((br, dout), lambda r, nb: (r * nblk + nb, 0)),
        out_shape=jax.ShapeDtypeStruct((_R * _NPAD, dout), jnp.float32),
    )(x, bases, comb.reshape(_R, 1, _NB))


def _tc_h(h, agg2, deg2, loop_w, bias, din, dout, last):
    nblk = 8
    br = _NPAD // nblk

    def body(h_ref, a_ref, d_ref, w_ref, b_ref, o_ref):
        agg = a_ref[0] + a_ref[1]
        deg = d_ref[0, :, 0:1] + d_ref[1, :, 0:1]
        norm = 1.0 / jnp.maximum(deg, 1.0)
        u = (agg * norm
             + jnp.dot(h_ref[...], w_ref[...], preferred_element_type=jnp.float32)
             + b_ref[...])
        if last:
            o_ref[...] = jax.nn.sigmoid(u)
        else:
            o_ref[...] = jnp.where(u >= 0.0, u, 0.01 * u)

    return pl.pallas_call(
        body,
        grid=(nblk,),
        in_specs=[
            pl.BlockSpec((br, din), lambda nb: (nb, 0)),
            pl.BlockSpec((2, br, dout), lambda nb: (0, nb, 0)),
            pl.BlockSpec((2, br, 16), lambda nb: (0, nb, 0)),
            pl.BlockSpec((din, dout), lambda nb: (0, 0)),
            pl.BlockSpec((1, dout), lambda nb: (0, 0)),
        ],
        out_specs=pl.BlockSpec((br, dout), lambda nb: (nb, 0)),
        out_shape=jax.ShapeDtypeStruct((_NPAD, dout), jnp.float32),
    )(h, agg2, deg2, loop_w, bias)


def _tc_conv(h3, kmat, cb):
    npix = _GRID * _GRID
    prows = npix + 2 * 104

    def body(h_ref, k_ref, cb_ref, o_ref, pad_ref):
        pad_ref[...] = jnp.zeros((prows, 32), jnp.float32)
        pad_ref[pl.ds(104, npix), :] = h_ref[pl.ds(0, npix), :]
        col = lax.broadcasted_iota(jnp.int32, (npix, 1), 0) % _GRID
        acc = jnp.zeros((npix, 9), jnp.float32)
        t = 0
        for dy in (-1, 0, 1):
            for dx in (-1, 0, 1):
                xs = pad_ref[pl.ds(104 + dy * _GRID + dx, npix), :]
                if dx == -1:
                    xs = jnp.where(col >= 1, xs, 0.0)
                elif dx == 1:
                    xs = jnp.where(col <= _GRID - 2, xs, 0.0)
                acc = acc + jnp.dot(xs, k_ref[t],
                                    preferred_element_type=jnp.float32)
                t += 1
        o_ref[...] = jax.nn.sigmoid(acc + cb_ref[0, 0])

    return pl.pallas_call(
        body,
        in_specs=[
            pl.BlockSpec((_NPAD, 32), lambda: (0, 0)),
            pl.BlockSpec((9, 32, 9), lambda: (0, 0, 0)),
            pl.BlockSpec((1, 1), lambda: (0, 0)),
        ],
        out_specs=pl.BlockSpec((npix, 9), lambda: (0, 0)),
        out_shape=jax.ShapeDtypeStruct((npix, 9), jnp.float32),
        scratch_shapes=[pltpu.VMEM((prows, 32), jnp.float32)],
    )(h3, kmat, cb)


def _phase_kernel(conv_w):
    cols = []
    zero = jnp.zeros((conv_w.shape[0],), conv_w.dtype)
    for dy in (-1, 0, 1):
        for dx in (-1, 0, 1):
            row = []
            for phy in range(3):
                for phx in range(3):
                    ky = 3 * dy + 4 - phy
                    kx = 3 * dx + 4 - phx
                    if 0 <= ky < 7 and 0 <= kx < 7:
                        row.append(conv_w[:, 0, 6 - ky, 6 - kx])
                    else:
                        row.append(zero)
            cols.append(jnp.stack(row, axis=1))
    return jnp.stack(cols, axis=0)


def kernel(features, etypes, src, dst,
           bases0, comb0, loop_w0, bias0,
           bases1, comb1, loop_w1, bias1,
           bases2, comb2, loop_w2, bias2,
           conv_w, conv_b):
    pad_e = _EP - _E
    shape3 = (_NW, _NCHUNK, _CH)
    srcp = jnp.concatenate([src.astype(jnp.int32),
                            jnp.zeros((pad_e,), jnp.int32)]).reshape(shape3)
    etyp = jnp.concatenate([etypes.astype(jnp.int32),
                            jnp.zeros((pad_e,), jnp.int32)]).reshape(shape3)
    dstp = jnp.concatenate([dst.astype(jnp.int32),
                            jnp.full((pad_e,), _NPAD - 1,
                                     jnp.int32)]).reshape(shape3)
    x = jnp.pad(features, ((0, _NPAD - _N), (0, 0)))

    zeros16 = jnp.zeros((_NPAD, 16), jnp.float32)
    zeros64 = jnp.zeros((_NPAD, 64), jnp.float32)
    zeros32 = jnp.zeros((_NPAD, 32), jnp.float32)

    gidx, deg2 = _sc_deg_gidx(srcp, etyp, dstp, zeros16)

    layers = [
        (bases0, comb0, loop_w0, bias0, 128, 64, zeros64),
        (bases1, comb1, loop_w1, bias1, 64, 64, zeros64),
        (bases2, comb2, loop_w2, bias2, 64, 32, zeros32),
    ]
    h = x
    for li, (bs, cm, lw, bi, din, dout, zz) in enumerate(layers):
        z = _tc_z(h, bs, cm, din, dout)
        agg2 = _sc_agg(z, gidx, dstp, zz, dout)
        h = _tc_h(h, agg2, deg2, lw, bi.reshape(1, dout), din, dout,
                  last=(li == 2))

    kmat = _phase_kernel(conv_w)
    y9 = _tc_conv(h, kmat, conv_b.reshape(1, 1))
    return (y9.reshape(_GRID, _GRID, 3, 3)
              .transpose(0, 2, 1, 3)
              .reshape(1, 3 * _GRID, 3 * _GRID))


# speedup vs baseline: 1.1531x; 1.1531x over previous
import functools

import jax
import jax.numpy as jnp
from jax import lax
from jax.experimental import pallas as pl
from jax.experimental.pallas import tpu as pltpu
from jax.experimental.pallas import tpu_sc as plsc

_N = 10000
_E = 320000
_R = 8
_NB = 4
_GRID = 100

_NPAD = 10112
_NW = 32
_EPT = 10240
_EP = _EPT * _NW
_CH = 128
_NCHUNK = _EPT // _CH
_RSUB = _NPAD // 16

def _mesh():
    return plsc.VectorSubcoreMesh(core_axis_name="c", subcore_axis_name="s",
                                  num_cores=2, num_subcores=16)


def _sc_deg_gidx(src3, ety3, dst3, zeros16):
    @functools.partial(
        pl.kernel,
        out_type=(
            jax.ShapeDtypeStruct((_NW, _NCHUNK, _CH), jnp.int32),
            jax.ShapeDtypeStruct((2, _NPAD, 16), jnp.float32),
        ),
        mesh=_mesh(),
        compiler_params=pltpu.CompilerParams(use_tc_tiling_on_sc=False),
        scratch_types=[
            pltpu.VMEM((_NCHUNK, _CH), jnp.int32),
            pltpu.VMEM((_NCHUNK, _CH), jnp.int32),
            pltpu.VMEM((_NCHUNK, _CH), jnp.int32),
            pltpu.VMEM((_NCHUNK, _CH), jnp.int32),
            pltpu.VMEM((_CH, 16), jnp.float32),
            pltpu.VMEM_SHARED((_NPAD, 16), jnp.float32),
            pltpu.SemaphoreType.DMA,
        ],
    )
    def k(src_h, ety_h, dst_h, z16_h, gidx_h, deg_h,
          src_v, ety_v, gidx_v, dst_v, ones_v, dacc, sem):
        c = lax.axis_index("c")
        s = lax.axis_index("s")
        wid = s * 2 + c
        rows = pl.ds(s * _RSUB, _RSUB)
        pltpu.sync_copy(z16_h.at[rows], dacc.at[rows])
        pltpu.sync_copy(src_h.at[wid], src_v)
        pltpu.sync_copy(ety_h.at[wid], ety_v)
        pltpu.sync_copy(dst_h.at[wid], dst_v)
        one16 = jnp.ones((16,), jnp.float32)

        def fill(i, _):
            ones_v[i, :] = one16
            return 0

        lax.fori_loop(0, _CH, fill, 0)

        def gix(i, _):
            for j in range(_CH // 16):
                sl = pl.ds(j * 16, 16)
                gidx_v[i, sl] = ety_v[i, sl] * _NPAD + src_v[i, sl]
            return 0

        lax.fori_loop(0, _NCHUNK, gix, 0)
        pltpu.sync_copy(gidx_v, gidx_h.at[wid])
        plsc.subcore_barrier()

        def step(g, _):
            pltpu.sync_copy(ones_v, dacc.at[dst_v.at[g]], add=True)
            return 0

        lax.fori_loop(0, _NCHUNK, step, 0)
        plsc.subcore_barrier()
        pltpu.sync_copy(dacc.at[rows], deg_h.at[c, rows])

    return k(src3, ety3, dst3, zeros16)


_NBUF = 4
_TOTCH = _EP // _CH
_CNT0 = 152
_CNT1 = (_TOTCH - 16 * _CNT0) // 16
_CNTMAX = max(_CNT0, _CNT1)


def _sc_agg(z, gidx3, dst3, zeros_d, dout):
    @functools.partial(
        pl.kernel,
        out_type=jax.ShapeDtypeStruct((2, _NPAD, dout), jnp.float32),
        mesh=_mesh(),
        compiler_params=pltpu.CompilerParams(use_tc_tiling_on_sc=False),
        scratch_types=[
            pltpu.VMEM((_CNTMAX, _CH), jnp.int32),
            pltpu.VMEM((_CNTMAX, _CH), jnp.int32),
            [pltpu.VMEM((_CH, dout), jnp.float32)] * _NBUF,
            [pltpu.SemaphoreType.DMA] * _NBUF,
            pltpu.VMEM_SHARED((_NPAD, dout), jnp.float32),
        ],
    )
    def k(z_h, gidx_h, dst_h, zz_h, agg_h, gidx_v, dst_v, rows_v, sems, acc):
        c = lax.axis_index("c")
        s = lax.axis_index("s")
        rows = pl.ds(s * _RSUB, _RSUB)
        pltpu.sync_copy(zz_h.at[rows], acc.at[rows])

        def gather(g, b):
            pltpu.async_copy(z_h.at[gidx_v.at[g]], rows_v[b], sems[b])

        def gwait(g, b):
            pltpu.make_async_copy(z_h.at[gidx_v.at[g]], rows_v[b],
                                  sems[b]).wait()

        def run(start, cnt):
            if cnt > 0:
                span = pl.ds(start, cnt)
                head = pl.ds(0, cnt)
                pltpu.sync_copy(gidx_h.at[span], gidx_v.at[head])
                pltpu.sync_copy(dst_h.at[span], dst_v.at[head])
            plsc.subcore_barrier()
            if cnt > 0:
                for b in range(_NBUF - 1):
                    gather(b, b)

                def outer(t, _):
                    for b in range(_NBUF):
                        g = t * _NBUF + b
                        nxt = g + _NBUF - 1
                        bn = (b + _NBUF - 1) % _NBUF

                        @pl.when(nxt < cnt)
                        def _():
                            gather(nxt, bn)

                        gwait(g, b)
                        pltpu.sync_copy(rows_v[b], acc.at[dst_v.at[g]],
                                        add=True)
                    return 0

                lax.fori_loop(0, cnt // _NBUF, outer, 0)
            plsc.subcore_barrier()
            pltpu.sync_copy(acc.at[rows], agg_h.at[c, rows])

        @pl.when(c == 0)
        def _():
            run(s * _CNT0, _CNT0)

        @pl.when(c == 1)
        def _():
            run(16 * _CNT0 + s * _CNT1, _CNT1)

    return k(z, gidx3.reshape(_TOTCH, _CH), dst3.reshape(_TOTCH, _CH),
             zeros_d)


def _tc_z(x, bases, comb, din, dout):
    nblk = 8
    br = _NPAD // nblk

    def body(x_ref, b_ref, c_ref, z_ref):
        w = (c_ref[0, 0, 0] * b_ref[0] + c_ref[0, 0, 1] * b_ref[1]
             + c_ref[0, 0, 2] * b_ref[2] + c_ref[0, 0, 3] * b_ref[3])
        z_ref[...] = jnp.dot(x_ref[...], w, preferred_element_type=jnp.float32)

    return pl.pallas_call(
        body,
        grid=(_R, nblk),
        in_specs=[
            pl.BlockSpec((br, din), lambda r, nb: (nb, 0)),
            pl.BlockSpec((_NB, din, dout), lambda r, nb: (0, 0, 0)),
            pl.BlockSpec((1, 1, _NB), lambda r, nb: (r, 0, 0)),
        ],
        out_specs=pl.BlockSpec((br, dout), lambda r, nb: (r * nblk + nb, 0)),
        out_shape=jax.ShapeDtypeStruct((_R * _NPAD, dout), jnp.float32),
    )(x, bases, comb.reshape(_R, 1, _NB))


def _tc_h(h, agg2, deg2, loop_w, bias, din, dout, last):
    nblk = 8
    br = _NPAD // nblk

    def body(h_ref, a_ref, d_ref, w_ref, b_ref, o_ref):
        agg = a_ref[0] + a_ref[1]
        deg = d_ref[0, :, 0:1] + d_ref[1, :, 0:1]
        norm = 1.0 / jnp.maximum(deg, 1.0)
        u = (agg * norm
             + jnp.dot(h_ref[...], w_ref[...], preferred_element_type=jnp.float32)
             + b_ref[...])
        if last:
            o_ref[...] = jax.nn.sigmoid(u)
        else:
            o_ref[...] = jnp.where(u >= 0.0, u, 0.01 * u)

    return pl.pallas_call(
        body,
        grid=(nblk,),
        in_specs=[
            pl.BlockSpec((br, din), lambda nb: (nb, 0)),
            pl.BlockSpec((2, br, dout), lambda nb: (0, nb, 0)),
            pl.BlockSpec((2, br, 16), lambda nb: (0, nb, 0)),
            pl.BlockSpec((din, dout), lambda nb: (0, 0)),
            pl.BlockSpec((1, dout), lambda nb: (0, 0)),
        ],
        out_specs=pl.BlockSpec((br, dout), lambda nb: (nb, 0)),
        out_shape=jax.ShapeDtypeStruct((_NPAD, dout), jnp.float32),
    )(h, agg2, deg2, loop_w, bias)


def _tc_conv(h3, kmat, cb):
    npix = _GRID * _GRID
    prows = npix + 2 * 104

    def body(h_ref, k_ref, cb_ref, o_ref, pad_ref):
        pad_ref[...] = jnp.zeros((prows, 32), jnp.float32)
        pad_ref[pl.ds(104, npix), :] = h_ref[pl.ds(0, npix), :]
        col = lax.broadcasted_iota(jnp.int32, (npix, 1), 0) % _GRID
        acc = jnp.zeros((npix, 9), jnp.float32)
        t = 0
        for dy in (-1, 0, 1):
            for dx in (-1, 0, 1):
                xs = pad_ref[pl.ds(104 + dy * _GRID + dx, npix), :]
                if dx == -1:
                    xs = jnp.where(col >= 1, xs, 0.0)
                elif dx == 1:
                    xs = jnp.where(col <= _GRID - 2, xs, 0.0)
                acc = acc + jnp.dot(xs, k_ref[t],
                                    preferred_element_type=jnp.float32)
                t += 1
        o_ref[...] = jax.nn.sigmoid(acc + cb_ref[0, 0])

    return pl.pallas_call(
        body,
        in_specs=[
            pl.BlockSpec((_NPAD, 32), lambda: (0, 0)),
            pl.BlockSpec((9, 32, 9), lambda: (0, 0, 0)),
            pl.BlockSpec((1, 1), lambda: (0, 0)),
        ],
        out_specs=pl.BlockSpec((npix, 9), lambda: (0, 0)),
        out_shape=jax.ShapeDtypeStruct((npix, 9), jnp.float32),
        scratch_shapes=[pltpu.VMEM((prows, 32), jnp.float32)],
    )(h3, kmat, cb)


def _phase_kernel(conv_w):
    cols = []
    zero = jnp.zeros((conv_w.shape[0],), conv_w.dtype)
    for dy in (-1, 0, 1):
        for dx in (-1, 0, 1):
            row = []
            for phy in range(3):
                for phx in range(3):
                    ky = 3 * dy + 4 - phy
                    kx = 3 * dx + 4 - phx
                    if 0 <= ky < 7 and 0 <= kx < 7:
                        row.append(conv_w[:, 0, 6 - ky, 6 - kx])
                    else:
                        row.append(zero)
            cols.append(jnp.stack(row, axis=1))
    return jnp.stack(cols, axis=0)


def kernel(features, etypes, src, dst,
           bases0, comb0, loop_w0, bias0,
           bases1, comb1, loop_w1, bias1,
           bases2, comb2, loop_w2, bias2,
           conv_w, conv_b):
    pad_e = _EP - _E
    shape3 = (_NW, _NCHUNK, _CH)
    srcp = jnp.concatenate([src.astype(jnp.int32),
                            jnp.zeros((pad_e,), jnp.int32)]).reshape(shape3)
    etyp = jnp.concatenate([etypes.astype(jnp.int32),
                            jnp.zeros((pad_e,), jnp.int32)]).reshape(shape3)
    dstp = jnp.concatenate([dst.astype(jnp.int32),
                            jnp.full((pad_e,), _NPAD - 1,
                                     jnp.int32)]).reshape(shape3)
    x = jnp.pad(features, ((0, _NPAD - _N), (0, 0)))

    zeros16 = jnp.zeros((_NPAD, 16), jnp.float32)
    zeros64 = jnp.zeros((_NPAD, 64), jnp.float32)
    zeros32 = jnp.zeros((_NPAD, 32), jnp.float32)

    gidx, deg2 = _sc_deg_gidx(srcp, etyp, dstp, zeros16)

    layers = [
        (bases0, comb0, loop_w0, bias0, 128, 64, zeros64),
        (bases1, comb1, loop_w1, bias1, 64, 64, zeros64),
        (bases2, comb2, loop_w2, bias2, 64, 32, zeros32),
    ]
    h = x
    for li, (bs, cm, lw, bi, din, dout, zz) in enumerate(layers):
        z = _tc_z(h, bs, cm, din, dout)
        agg2 = _sc_agg(z, gidx, dstp, zz, dout)
        h = _tc_h(h, agg2, deg2, lw, bi.reshape(1, dout), din, dout,
                  last=(li == 2))

    kmat = _phase_kernel(conv_w)
    y9 = _tc_conv(h, kmat, conv_b.reshape(1, 1))
    return (y9.reshape(_GRID, _GRID, 3, 3)
              .transpose(0, 2, 1, 3)
              .reshape(1, 3 * _GRID, 3 * _GRID))


# speedup vs baseline: 1.6377x; 1.4203x over previous
import functools

import jax
import jax.numpy as jnp
from jax import lax
from jax.experimental import pallas as pl
from jax.experimental.pallas import tpu as pltpu
from jax.experimental.pallas import tpu_sc as plsc

_N = 10000
_E = 320000
_R = 8
_NB = 4
_GRID = 100

_NPAD = 10112
_NW = 32
_EPT = 10240
_EP = _EPT * _NW
_CH = 128
_NCHUNK = _EPT // _CH
_RSUB = _NPAD // 16

def _mesh():
    return plsc.VectorSubcoreMesh(core_axis_name="c", subcore_axis_name="s",
                                  num_cores=2, num_subcores=16)


def _sc_deg_gidx(src3, ety3, dst3, zeros16):
    @functools.partial(
        pl.kernel,
        out_type=(
            jax.ShapeDtypeStruct((_NW, _NCHUNK, _CH), jnp.int32),
            jax.ShapeDtypeStruct((2, _NPAD, 16), jnp.float32),
        ),
        mesh=_mesh(),
        compiler_params=pltpu.CompilerParams(use_tc_tiling_on_sc=False),
        scratch_types=[
            pltpu.VMEM((_NCHUNK, _CH), jnp.int32),
            pltpu.VMEM((_NCHUNK, _CH), jnp.int32),
            pltpu.VMEM((_NCHUNK, _CH), jnp.int32),
            pltpu.VMEM((_NCHUNK, _CH), jnp.int32),
            pltpu.VMEM((_CH, 16), jnp.float32),
            pltpu.VMEM_SHARED((_NPAD, 16), jnp.float32),
            pltpu.SemaphoreType.DMA,
        ],
    )
    def k(src_h, ety_h, dst_h, z16_h, gidx_h, deg_h,
          src_v, ety_v, gidx_v, dst_v, ones_v, dacc, sem):
        c = lax.axis_index("c")
        s = lax.axis_index("s")
        wid = s * 2 + c
        rows = pl.ds(s * _RSUB, _RSUB)
        pltpu.sync_copy(z16_h.at[rows], dacc.at[rows])
        pltpu.sync_copy(src_h.at[wid], src_v)
        pltpu.sync_copy(ety_h.at[wid], ety_v)
        pltpu.sync_copy(dst_h.at[wid], dst_v)
        one16 = jnp.ones((16,), jnp.float32)

        def fill(i, _):
            ones_v[i, :] = one16
            return 0

        lax.fori_loop(0, _CH, fill, 0)

        def gix(i, _):
            for j in range(_CH // 16):
                sl = pl.ds(j * 16, 16)
                gidx_v[i, sl] = ety_v[i, sl] * _NPAD + src_v[i, sl]
            return 0

        lax.fori_loop(0, _NCHUNK, gix, 0)
        pltpu.sync_copy(gidx_v, gidx_h.at[wid])
        plsc.subcore_barrier()

        def step(g, _):
            pltpu.sync_copy(ones_v, dacc.at[dst_v.at[g]], add=True)
            return 0

        lax.fori_loop(0, _NCHUNK, step, 0)
        plsc.subcore_barrier()
        pltpu.sync_copy(dacc.at[rows], deg_h.at[c, rows])

    return k(src3, ety3, dst3, zeros16)


_NBUF = 4
_TOTCH = _EP // _CH
_CNT0 = 152
_CNT1 = (_TOTCH - 16 * _CNT0) // 16
_CNTMAX = max(_CNT0, _CNT1)


def _sc_agg(z, gidx3, dst3, zeros_d, dout):
    @functools.partial(
        pl.kernel,
        out_type=jax.ShapeDtypeStruct((2, _NPAD, dout), jnp.float32),
        mesh=_mesh(),
        compiler_params=pltpu.CompilerParams(use_tc_tiling_on_sc=False),
        scratch_types=[
            pltpu.VMEM((_CNTMAX, _CH), jnp.int32),
            pltpu.VMEM((_CNTMAX, _CH), jnp.int32),
            [pltpu.VMEM((_CH, dout), jnp.float32)] * _NBUF,
            [pltpu.SemaphoreType.DMA] * _NBUF,
            pltpu.VMEM_SHARED((_NPAD, dout), jnp.float32),
        ],
    )
    def k(z_h, gidx_h, dst_h, zz_h, agg_h, gidx_v, dst_v, rows_v, sems, acc):
        c = lax.axis_index("c")
        s = lax.axis_index("s")
        rows = pl.ds(s * _RSUB, _RSUB)
        pltpu.sync_copy(zz_h.at[rows], acc.at[rows])

        def gather(g, b):
            pltpu.async_copy(z_h.at[gidx_v.at[g]], rows_v[b], sems[b])

        def gwait(g, b):
            pltpu.make_async_copy(z_h.at[gidx_v.at[g]], rows_v[b],
                                  sems[b]).wait()

        def run(start, cnt):
            if cnt > 0:
                span = pl.ds(start, cnt)
                head = pl.ds(0, cnt)
                pltpu.sync_copy(gidx_h.at[span], gidx_v.at[head])
                pltpu.sync_copy(dst_h.at[span], dst_v.at[head])
            plsc.subcore_barrier()
            if cnt > 0:
                for b in range(_NBUF - 1):
                    gather(b, b)

                def outer(t, _):
                    for b in range(_NBUF):
                        g = t * _NBUF + b
                        nxt = g + _NBUF - 1
                        bn = (b + _NBUF - 1) % _NBUF

                        @pl.when(nxt < cnt)
                        def _():
                            gather(nxt, bn)

                        gwait(g, b)
                        pltpu.sync_copy(rows_v[b], acc.at[dst_v.at[g]],
                                        add=True)
                    return 0

                lax.fori_loop(0, cnt // _NBUF, outer, 0)
            plsc.subcore_barrier()
            pltpu.sync_copy(acc.at[rows], agg_h.at[c, rows])

        @pl.when(c == 0)
        def _():
            run(s * _CNT0, _CNT0)

        @pl.when(c == 1)
        def _():
            run(16 * _CNT0 + s * _CNT1, _CNT1)

    return k(z, gidx3.reshape(_TOTCH, _CH), dst3.reshape(_TOTCH, _CH),
             zeros_d)


def _tc_z(x, bases, comb, din, dout):
    nblk = 8
    br = _NPAD // nblk

    def body(x_ref, b_ref, c_ref, z_ref):
        w = (c_ref[0, 0, 0] * b_ref[0] + c_ref[0, 0, 1] * b_ref[1]
             + c_ref[0, 0, 2] * b_ref[2] + c_ref[0, 0, 3] * b_ref[3])
        z_ref[...] = jnp.dot(x_ref[...], w, preferred_element_type=jnp.float32)

    return pl.pallas_call(
        body,
        grid=(_R, nblk),
        in_specs=[
            pl.BlockSpec((br, din), lambda r, nb: (nb, 0)),
            pl.BlockSpec((_NB, din, dout), lambda r, nb: (0, 0, 0)),
            pl.BlockSpec((1, 1, _NB), lambda r, nb: (r, 0, 0)),
        ],
        out_specs=pl.BlockSpec((br, dout), lambda r, nb: (r * nblk + nb, 0)),
        out_shape=jax.ShapeDtypeStruct((_R * _NPAD, dout), jnp.float32),
    )(x, bases, comb.reshape(_R, 1, _NB))


def _tc_h(h, agg2, deg2, loop_w, bias, din, dout, last):
    nblk = 8
    br = _NPAD // nblk

    def body(h_ref, a_ref, d_ref, w_ref, b_ref, o_ref):
        agg = a_ref[0] + a_ref[1]
        deg = d_ref[0, :, 0:1] + d_ref[1, :, 0:1]
        norm = 1.0 / jnp.maximum(deg, 1.0)
        u = (agg * norm
             + jnp.dot(h_ref[...], w_ref[...], preferred_element_type=jnp.float32)
             + b_ref[...])
        if last:
            o_ref[...] = jax.nn.sigmoid(u)
        else:
            o_ref[...] = jnp.where(u >= 0.0, u, 0.01 * u)

    return pl.pallas_call(
        body,
        grid=(nblk,),
        in_specs=[
            pl.BlockSpec((br, din), lambda nb: (nb, 0)),
            pl.BlockSpec((2, br, dout), lambda nb: (0, nb, 0)),
            pl.BlockSpec((2, br, 16), lambda nb: (0, nb, 0)),
            pl.BlockSpec((din, dout), lambda nb: (0, 0)),
            pl.BlockSpec((1, dout), lambda nb: (0, 0)),
        ],
        out_specs=pl.BlockSpec((br, dout), lambda nb: (nb, 0)),
        out_shape=jax.ShapeDtypeStruct((_NPAD, dout), jnp.float32),
    )(h, agg2, deg2, loop_w, bias)


def _tc_conv(h3, kmat, cb):
    npix = _GRID * _GRID
    prows = npix + 2 * 104

    def body(h_ref, k_ref, cb_ref, o_ref, pad_ref):
        pad_ref[...] = jnp.zeros((prows, 32), jnp.float32)
        pad_ref[pl.ds(104, npix), :] = h_ref[pl.ds(0, npix), :]
        col = lax.broadcasted_iota(jnp.int32, (npix, 1), 0) % _GRID
        acc = jnp.zeros((npix, 9), jnp.float32)
        t = 0
        for dy in (-1, 0, 1):
            for dx in (-1, 0, 1):
                xs = pad_ref[pl.ds(104 + dy * _GRID + dx, npix), :]
                if dx == -1:
                    xs = jnp.where(col >= 1, xs, 0.0)
                elif dx == 1:
                    xs = jnp.where(col <= _GRID - 2, xs, 0.0)
                acc = acc + jnp.dot(xs, k_ref[t],
                                    preferred_element_type=jnp.float32)
                t += 1
        o_ref[...] = jax.nn.sigmoid(acc + cb_ref[0, 0])

    return pl.pallas_call(
        body,
        in_specs=[
            pl.BlockSpec((_NPAD, 32), lambda: (0, 0)),
            pl.BlockSpec((9, 32, 9), lambda: (0, 0, 0)),
            pl.BlockSpec((1, 1), lambda: (0, 0)),
        ],
        out_specs=pl.BlockSpec((npix, 9), lambda: (0, 0)),
        out_shape=jax.ShapeDtypeStruct((npix, 9), jnp.float32),
        scratch_shapes=[pltpu.VMEM((prows, 32), jnp.float32)],
    )(h3, kmat, cb)


def _phase_kernel(conv_w):
    cols = []
    zero = jnp.zeros((conv_w.shape[0],), conv_w.dtype)
    for dy in (-1, 0, 1):
        for dx in (-1, 0, 1):
            row = []
            for phy in range(3):
                for phx in range(3):
                    ky = 3 * dy + 4 - phy
                    kx = 3 * dx + 4 - phx
                    if 0 <= ky < 7 and 0 <= kx < 7:
                        row.append(conv_w[:, 0, 6 - ky, 6 - kx])
                    else:
                        row.append(zero)
            cols.append(jnp.stack(row, axis=1))
    return jnp.stack(cols, axis=0)


def kernel(features, etypes, src, dst,
           bases0, comb0, loop_w0, bias0,
           bases1, comb1, loop_w1, bias1,
           bases2, comb2, loop_w2, bias2,
           conv_w, conv_b):
    pad_e = _EP - _E
    shape3 = (_NW, _NCHUNK, _CH)
    pad_i = jnp.arange(pad_e, dtype=jnp.int32)
    srcp = jnp.concatenate([src.astype(jnp.int32),
                            pad_i % _N]).reshape(shape3)
    etyp = jnp.concatenate([etypes.astype(jnp.int32),
                            jnp.zeros((pad_e,), jnp.int32)]).reshape(shape3)
    dstp = jnp.concatenate([dst.astype(jnp.int32),
                            _N + pad_i % (_NPAD - _N)]).reshape(shape3)
    x = jnp.pad(features, ((0, _NPAD - _N), (0, 0)))

    zeros16 = jnp.zeros((_NPAD, 16), jnp.float32)
    zeros64 = jnp.zeros((_NPAD, 64), jnp.float32)
    zeros32 = jnp.zeros((_NPAD, 32), jnp.float32)

    gidx, deg2 = _sc_deg_gidx(srcp, etyp, dstp, zeros16)

    layers = [
        (bases0, comb0, loop_w0, bias0, 128, 64, zeros64),
        (bases1, comb1, loop_w1, bias1, 64, 64, zeros64),
        (bases2, comb2, loop_w2, bias2, 64, 32, zeros32),
    ]
    h = x
    for li, (bs, cm, lw, bi, din, dout, zz) in enumerate(layers):
        z = _tc_z(h, bs, cm, din, dout)
        agg2 = _sc_agg(z, gidx, dstp, zz, dout)
        h = _tc_h(h, agg2, deg2, lw, bi.reshape(1, dout), din, dout,
                  last=(li == 2))

    kmat = _phase_kernel(conv_w)
    y9 = _tc_conv(h, kmat, conv_b.reshape(1, 1))
    return (y9.reshape(_GRID, _GRID, 3, 3)
              .transpose(0, 2, 1, 3)
              .reshape(1, 3 * _GRID, 3 * _GRID))


# speedup vs baseline: 1.8993x; 1.1597x over previous
import functools

import jax
import jax.numpy as jnp
from jax import lax
from jax.experimental import pallas as pl
from jax.experimental.pallas import tpu as pltpu
from jax.experimental.pallas import tpu_sc as plsc

_N = 10000
_E = 320000
_R = 8
_NB = 4
_GRID = 100

_NPAD = 10112
_NW = 32
_EPT = 10240
_EP = _EPT * _NW
_CH = 128
_NCHUNK = _EPT // _CH
_RSUB = _NPAD // 16

def _mesh():
    return plsc.VectorSubcoreMesh(core_axis_name="c", subcore_axis_name="s",
                                  num_cores=2, num_subcores=16)


def _sc_deg_gidx(src3, ety3, dst3, zeros16):
    @functools.partial(
        pl.kernel,
        out_type=(
            jax.ShapeDtypeStruct((_NW, _NCHUNK, _CH), jnp.int32),
            jax.ShapeDtypeStruct((2, _NPAD, 16), jnp.float32),
        ),
        mesh=_mesh(),
        compiler_params=pltpu.CompilerParams(use_tc_tiling_on_sc=False),
        scratch_types=[
            pltpu.VMEM((_NCHUNK, _CH), jnp.int32),
            pltpu.VMEM((_NCHUNK, _CH), jnp.int32),
            pltpu.VMEM((_NCHUNK, _CH), jnp.int32),
            pltpu.VMEM((_NCHUNK, _CH), jnp.int32),
            pltpu.VMEM((_CH, 16), jnp.float32),
            pltpu.VMEM_SHARED((_NPAD, 16), jnp.float32),
            pltpu.SemaphoreType.DMA,
        ],
    )
    def k(src_h, ety_h, dst_h, z16_h, gidx_h, deg_h,
          src_v, ety_v, gidx_v, dst_v, ones_v, dacc, sem):
        c = lax.axis_index("c")
        s = lax.axis_index("s")
        wid = s * 2 + c
        rows = pl.ds(s * _RSUB, _RSUB)
        pltpu.sync_copy(z16_h.at[rows], dacc.at[rows])
        pltpu.sync_copy(src_h.at[wid], src_v)
        pltpu.sync_copy(ety_h.at[wid], ety_v)
        pltpu.sync_copy(dst_h.at[wid], dst_v)
        one16 = jnp.ones((16,), jnp.float32)

        def fill(i, _):
            ones_v[i, :] = one16
            return 0

        lax.fori_loop(0, _CH, fill, 0)

        def gix(i, _):
            for j in range(_CH // 16):
                sl = pl.ds(j * 16, 16)
                gidx_v[i, sl] = ety_v[i, sl] * _NPAD + src_v[i, sl]
            return 0

        lax.fori_loop(0, _NCHUNK, gix, 0)
        pltpu.sync_copy(gidx_v, gidx_h.at[wid])
        plsc.subcore_barrier()

        def step(g, _):
            pltpu.sync_copy(ones_v, dacc.at[dst_v.at[g]], add=True)
            return 0

        lax.fori_loop(0, _NCHUNK, step, 0)
        plsc.subcore_barrier()
        pltpu.sync_copy(dacc.at[rows], deg_h.at[c, rows])

    return k(src3, ety3, dst3, zeros16)


_NBUF = 4
_TOTCH = _EP // _CH
_CNT0 = 80
_CNT1 = (_TOTCH - 16 * _CNT0) // 16
_CNTMAX = max(_CNT0, _CNT1)


def _sc_agg(z, gidx3, dst3, zeros_d, dout):
    @functools.partial(
        pl.kernel,
        out_type=jax.ShapeDtypeStruct((2, _NPAD, dout), jnp.float32),
        mesh=_mesh(),
        compiler_params=pltpu.CompilerParams(use_tc_tiling_on_sc=False),
        scratch_types=[
            pltpu.VMEM((_CNTMAX, _CH), jnp.int32),
            pltpu.VMEM((_CNTMAX, _CH), jnp.int32),
            [pltpu.VMEM((_CH, dout), jnp.float32)] * _NBUF,
            [pltpu.SemaphoreType.DMA] * _NBUF,
            pltpu.VMEM_SHARED((_NPAD, dout), jnp.float32),
        ],
    )
    def k(z_h, gidx_h, dst_h, zz_h, agg_h, gidx_v, dst_v, rows_v, sems, acc):
        c = lax.axis_index("c")
        s = lax.axis_index("s")
        rows = pl.ds(s * _RSUB, _RSUB)
        pltpu.sync_copy(zz_h.at[rows], acc.at[rows])

        def gather(g, b):
            pltpu.async_copy(z_h.at[gidx_v.at[g]], rows_v[b], sems[b])

        def gwait(g, b):
            pltpu.make_async_copy(z_h.at[gidx_v.at[g]], rows_v[b],
                                  sems[b]).wait()

        def run(start, cnt):
            if cnt > 0:
                span = pl.ds(start, cnt)
                head = pl.ds(0, cnt)
                pltpu.sync_copy(gidx_h.at[span], gidx_v.at[head])
                pltpu.sync_copy(dst_h.at[span], dst_v.at[head])
            plsc.subcore_barrier()
            if cnt > 0:
                for b in range(_NBUF - 1):
                    gather(b, b)

                def outer(t, _):
                    for b in range(_NBUF):
                        g = t * _NBUF + b
                        nxt = g + _NBUF - 1
                        bn = (b + _NBUF - 1) % _NBUF

                        @pl.when(nxt < cnt)
                        def _():
                            gather(nxt, bn)

                        gwait(g, b)
                        pltpu.sync_copy(rows_v[b], acc.at[dst_v.at[g]],
                                        add=True)
                    return 0

                lax.fori_loop(0, cnt // _NBUF, outer, 0)
            plsc.subcore_barrier()
            pltpu.sync_copy(acc.at[rows], agg_h.at[c, rows])

        @pl.when(c == 0)
        def _():
            run(s * _CNT0, _CNT0)

        @pl.when(c == 1)
        def _():
            run(16 * _CNT0 + s * _CNT1, _CNT1)

    return k(z, gidx3.reshape(_TOTCH, _CH), dst3.reshape(_TOTCH, _CH),
             zeros_d)


def _tc_z(x, bases, comb, din, dout):
    nblk = 8
    br = _NPAD // nblk

    def body(x_ref, b_ref, c_ref, z_ref):
        w = (c_ref[0, 0, 0] * b_ref[0] + c_ref[0, 0, 1] * b_ref[1]
             + c_ref[0, 0, 2] * b_ref[2] + c_ref[0, 0, 3] * b_ref[3])
        z_ref[...] = jnp.dot(x_ref[...], w, preferred_element_type=jnp.float32)

    return pl.pallas_call(
        body,
        grid=(_R, nblk),
        in_specs=[
            pl.BlockSpec((br, din), lambda r, nb: (nb, 0)),
            pl.BlockSpec((_NB, din, dout), lambda r, nb: (0, 0, 0)),
            pl.BlockSpec((1, 1, _NB), lambda r, nb: (r, 0, 0)),
        ],
        out_specs=pl.BlockSpec((br, dout), lambda r, nb: (r * nblk + nb, 0)),
        out_shape=jax.ShapeDtypeStruct((_R * _NPAD, dout), jnp.float32),
    )(x, bases, comb.reshape(_R, 1, _NB))


def _tc_h(h, agg2, deg2, loop_w, bias, din, dout, last):
    nblk = 8
    br = _NPAD // nblk

    def body(h_ref, a_ref, d_ref, w_ref, b_ref, o_ref):
        agg = a_ref[0] + a_ref[1]
        deg = d_ref[0, :, 0:1] + d_ref[1, :, 0:1]
        norm = 1.0 / jnp.maximum(deg, 1.0)
        u = (agg * norm
             + jnp.dot(h_ref[...], w_ref[...], preferred_element_type=jnp.float32)
             + b_ref[...])
        if last:
            o_ref[...] = jax.nn.sigmoid(u)
        else:
            o_ref[...] = jnp.where(u >= 0.0, u, 0.01 * u)

    return pl.pallas_call(
        body,
        grid=(nblk,),
        in_specs=[
            pl.BlockSpec((br, din), lambda nb: (nb, 0)),
            pl.BlockSpec((2, br, dout), lambda nb: (0, nb, 0)),
            pl.BlockSpec((2, br, 16), lambda nb: (0, nb, 0)),
            pl.BlockSpec((din, dout), lambda nb: (0, 0)),
            pl.BlockSpec((1, dout), lambda nb: (0, 0)),
        ],
        out_specs=pl.BlockSpec((br, dout), lambda nb: (nb, 0)),
        out_shape=jax.ShapeDtypeStruct((_NPAD, dout), jnp.float32),
    )(h, agg2, deg2, loop_w, bias)


def _tc_conv(h3, kmat, cb):
    npix = _GRID * _GRID
    prows = npix + 2 * 104

    def body(h_ref, k_ref, cb_ref, o_ref, pad_ref):
        pad_ref[...] = jnp.zeros((prows, 32), jnp.float32)
        pad_ref[pl.ds(104, npix), :] = h_ref[pl.ds(0, npix), :]
        col = lax.broadcasted_iota(jnp.int32, (npix, 1), 0) % _GRID
        acc = jnp.zeros((npix, 9), jnp.float32)
        t = 0
        for dy in (-1, 0, 1):
            for dx in (-1, 0, 1):
                xs = pad_ref[pl.ds(104 + dy * _GRID + dx, npix), :]
                if dx == -1:
                    xs = jnp.where(col >= 1, xs, 0.0)
                elif dx == 1:
                    xs = jnp.where(col <= _GRID - 2, xs, 0.0)
                acc = acc + jnp.dot(xs, k_ref[t],
                                    preferred_element_type=jnp.float32)
                t += 1
        o_ref[...] = jax.nn.sigmoid(acc + cb_ref[0, 0])

    return pl.pallas_call(
        body,
        in_specs=[
            pl.BlockSpec((_NPAD, 32), lambda: (0, 0)),
            pl.BlockSpec((9, 32, 9), lambda: (0, 0, 0)),
            pl.BlockSpec((1, 1), lambda: (0, 0)),
        ],
        out_specs=pl.BlockSpec((npix, 9), lambda: (0, 0)),
        out_shape=jax.ShapeDtypeStruct((npix, 9), jnp.float32),
        scratch_shapes=[pltpu.VMEM((prows, 32), jnp.float32)],
    )(h3, kmat, cb)


def _phase_kernel(conv_w):
    cols = []
    zero = jnp.zeros((conv_w.shape[0],), conv_w.dtype)
    for dy in (-1, 0, 1):
        for dx in (-1, 0, 1):
            row = []
            for phy in range(3):
                for phx in range(3):
                    ky = 3 * dy + 4 - phy
                    kx = 3 * dx + 4 - phx
                    if 0 <= ky < 7 and 0 <= kx < 7:
                        row.append(conv_w[:, 0, 6 - ky, 6 - kx])
                    else:
                        row.append(zero)
            cols.append(jnp.stack(row, axis=1))
    return jnp.stack(cols, axis=0)


def kernel(features, etypes, src, dst,
           bases0, comb0, loop_w0, bias0,
           bases1, comb1, loop_w1, bias1,
           bases2, comb2, loop_w2, bias2,
           conv_w, conv_b):
    pad_e = _EP - _E
    shape3 = (_NW, _NCHUNK, _CH)
    pad_i = jnp.arange(pad_e, dtype=jnp.int32)
    srcp = jnp.concatenate([src.astype(jnp.int32),
                            pad_i % _N]).reshape(shape3)
    etyp = jnp.concatenate([etypes.astype(jnp.int32),
                            jnp.zeros((pad_e,), jnp.int32)]).reshape(shape3)
    dstp = jnp.concatenate([dst.astype(jnp.int32),
                            _N + pad_i % (_NPAD - _N)]).reshape(shape3)
    x = jnp.pad(features, ((0, _NPAD - _N), (0, 0)))

    zeros16 = jnp.zeros((_NPAD, 16), jnp.float32)
    zeros64 = jnp.zeros((_NPAD, 64), jnp.float32)
    zeros32 = jnp.zeros((_NPAD, 32), jnp.float32)

    gidx, deg2 = _sc_deg_gidx(srcp, etyp, dstp, zeros16)

    layers = [
        (bases0, comb0, loop_w0, bias0, 128, 64, zeros64),
        (bases1, comb1, loop_w1, bias1, 64, 64, zeros64),
        (bases2, comb2, loop_w2, bias2, 64, 32, zeros32),
    ]
    h = x
    for li, (bs, cm, lw, bi, din, dout, zz) in enumerate(layers):
        z = _tc_z(h, bs, cm, din, dout)
        agg2 = _sc_agg(z, gidx, dstp, zz, dout)
        h = _tc_h(h, agg2, deg2, lw, bi.reshape(1, dout), din, dout,
                  last=(li == 2))

    kmat = _phase_kernel(conv_w)
    y9 = _tc_conv(h, kmat, conv_b.reshape(1, 1))
    return (y9.reshape(_GRID, _GRID, 3, 3)
              .transpose(0, 2, 1, 3)
              .reshape(1, 3 * _GRID, 3 * _GRID))
